# baseline (speedup 1.0000x reference)
import functools

import jax
import jax.numpy as jnp
from jax import lax
from jax.experimental import pallas as pl
from jax.experimental.pallas import tpu as pltpu
from jax.experimental.pallas import tpu_sc as plsc

N = 10000
D = 128
E = 320000
G = 64
NCORES = 2
NSUB = 16
NW = NCORES * NSUB
K = 128
GS = 20
NG = 4
NGP = NG + 1
NBUF = 2
NPAD = 10112
RPT = NPAD // NSUB
BN_EPS_K = 1e-5


def _sc_agg_body(h_hbm, col_hbm, row_hbm, zeros_hbm, out_hbm,
                 idxc_v, idxr_v, rows_v, gsem, isemc, isemr, accum_sh):
    c = lax.axis_index("c")
    s = lax.axis_index("s")
    wid = c * NSUB + s
    pltpu.sync_copy(zeros_hbm, accum_sh.at[pl.ds(s * RPT, RPT)])
    pltpu.sync_copy(col_hbm.at[wid, 0], idxc_v.at[0])
    pltpu.sync_copy(row_hbm.at[wid, 0], idxr_v.at[0])
    plsc.subcore_barrier()
    for b in range(NBUF):
        pltpu.async_copy(h_hbm.at[idxc_v.at[0, b]], rows_v.at[b], gsem.at[b])

    def group(g, carry):
        p = lax.rem(g, 2)
        q = 1 - p
        pltpu.async_copy(col_hbm.at[wid, g + 1], idxc_v.at[q], isemc.at[q])
        pltpu.async_copy(row_hbm.at[wid, g + 1], idxr_v.at[q], isemr.at[q])
        for b in range(GS):
            slot = b % NBUF
            pltpu.make_async_copy(h_hbm.at[idxc_v.at[p, b]], rows_v.at[slot],
                                  gsem.at[slot]).wait()
            pltpu.sync_copy(rows_v.at[slot], accum_sh.at[idxr_v.at[p, b]],
                            add=True)
            if b + NBUF < GS:
                pltpu.async_copy(h_hbm.at[idxc_v.at[p, b + NBUF]],
                                 rows_v.at[slot], gsem.at[slot])
        pltpu.make_async_copy(col_hbm.at[wid, g + 1], idxc_v.at[q],
                              isemc.at[q]).wait()
        pltpu.make_async_copy(row_hbm.at[wid, g + 1], idxr_v.at[q],
                              isemr.at[q]).wait()
        for b in range(NBUF):
            pltpu.async_copy(h_hbm.at[idxc_v.at[q, b]], rows_v.at[b],
                             gsem.at[b])
        return carry

    lax.fori_loop(0, NG, group, 0)
    for b in range(NBUF):
        pltpu.make_async_copy(h_hbm.at[idxc_v.at[0, b]], rows_v.at[b],
                              gsem.at[b]).wait()
    plsc.subcore_barrier()
    pltpu.sync_copy(accum_sh.at[pl.ds(s * RPT, RPT)],
                    out_hbm.at[c].at[pl.ds(s * RPT, RPT)])


_sc_agg = pl.kernel(
    _sc_agg_body,
    out_type=jax.ShapeDtypeStruct((NCORES, NPAD, D), jnp.float32),
    mesh=plsc.VectorSubcoreMesh(core_axis_name="c", subcore_axis_name="s"),
    scratch_types=[
        pltpu.MemorySpace.VMEM((2, GS, K), jnp.int32),
        pltpu.MemorySpace.VMEM((2, GS, K), jnp.int32),
        pltpu.MemorySpace.VMEM((NBUF, K, D), jnp.float32),
        pltpu.SemaphoreType.DMA((NBUF,)),
        pltpu.SemaphoreType.DMA((2,)),
        pltpu.SemaphoreType.DMA((2,)),
        pltpu.MemorySpace.VMEM_SHARED((NPAD, D), jnp.float32),
    ],
)


def _bn_relu(z, gamma, beta):
    mean = jnp.mean(z, axis=0, keepdims=True)
    var = jnp.mean((z - mean) * (z - mean), axis=0, keepdims=True)
    zn = gamma * (z - mean) * lax.rsqrt(var + BN_EPS_K) + beta
    return jnp.maximum(zn, 0.0)


def _tc_layer_body(eps_ref, parts_ref, h_ref,
                   w0_ref, b0_ref, g0_ref, be0_ref,
                   w1_ref, b1_ref, gl_ref, bel_ref, out_ref):
    eps = eps_ref[0]
    pooled = (parts_ref[0, 0:N, :] + parts_ref[1, 0:N, :]
              + (1.0 + eps) * h_ref[...])
    z = lax.dot_general(pooled, w0_ref[...], (((1,), (1,)), ((), ())),
                        preferred_element_type=jnp.float32) + b0_ref[...]
    z = _bn_relu(z, g0_ref[...], be0_ref[...])
    z = lax.dot_general(z, w1_ref[...], (((1,), (1,)), ((), ())),
                        preferred_element_type=jnp.float32) + b1_ref[...]
    out_ref[...] = _bn_relu(z, gl_ref[...], bel_ref[...])


_tc_layer = pl.pallas_call(
    _tc_layer_body,
    out_shape=jax.ShapeDtypeStruct((N, D), jnp.float32),
    in_specs=[
        pl.BlockSpec(memory_space=pltpu.MemorySpace.SMEM),
    ] + [pl.BlockSpec(memory_space=pltpu.MemorySpace.VMEM)] * 10,
    out_specs=pl.BlockSpec(memory_space=pltpu.MemorySpace.VMEM),
)


def _tc_pool_body(batch_ref, h0_ref, h1_ref, h2_ref, h3_ref, h4_ref,
                  w_ref, b_ref, out_ref):
    b = jnp.broadcast_to(batch_ref[...], (G, N))
    gi = lax.broadcasted_iota(jnp.int32, (G, N), 0)
    p = (b == gi).astype(jnp.float32)
    counts = jnp.sum(p, axis=1, keepdims=True)
    inv = 1.0 / jnp.maximum(counts, 1.0)
    acc = jnp.zeros((G, D), jnp.float32)
    for l in range(5):
        h_ref = (h0_ref, h1_ref, h2_ref, h3_ref, h4_ref)[l]
        pooled = lax.dot_general(p, h_ref[...], (((1,), (0,)), ((), ())),
                                 preferred_element_type=jnp.float32) * inv
        acc = acc + lax.dot_general(pooled, w_ref[l],
                                    (((1,), (1,)), ((), ())),
                                    preferred_element_type=jnp.float32)
        acc = acc + b_ref[l]
    out_ref[...] = acc


_tc_pool = pl.pallas_call(
    _tc_pool_body,
    out_shape=jax.ShapeDtypeStruct((G, D), jnp.float32),
)


def kernel(x, params, edge_index, batch):
    row = edge_index[0]
    col = edge_index[1]
    pad = NW * NG * GS * K - E
    colp = jnp.concatenate([col, jnp.zeros((pad,), jnp.int32)]).reshape(NW, NG, GS, K)
    rowp = jnp.concatenate([row, jnp.full((pad,), N, jnp.int32)]).reshape(NW, NG, GS, K)
    colp = jnp.concatenate([colp, jnp.zeros((NW, 1, GS, K), jnp.int32)], axis=1)
    rowp = jnp.concatenate([rowp, jnp.full((NW, 1, GS, K), N, jnp.int32)], axis=1)
    zeros_init = jnp.zeros((RPT, D), jnp.float32)

    hs = [x]
    h = x
    for l in range(4):
        parts = _sc_agg(h, colp, rowp, zeros_init)
        mlp = params["mlp%d" % l]
        h = _tc_layer(
            params["eps"][l].reshape(1),
            parts, h,
            mlp["W0"], mlp["b0"].reshape(1, D),
            mlp["bn_g0"].reshape(1, D), mlp["bn_b0"].reshape(1, D),
            mlp["W1"], mlp["b1"].reshape(1, D),
            params["bn_g%d" % l].reshape(1, D),
            params["bn_b%d" % l].reshape(1, D),
        )
        hs.append(h)

    wstack = jnp.stack([params["pred%d_W" % l] for l in range(5)])
    bstack = jnp.stack([params["pred%d_b" % l] for l in range(5)]).reshape(5, 1, D)
    score = _tc_pool(batch.reshape(1, N), *hs, wstack, bstack)
    return score


# speedup vs baseline: 1.6718x; 1.6718x over previous
import functools

import jax
import jax.numpy as jnp
from jax import lax
from jax.experimental import pallas as pl
from jax.experimental.pallas import tpu as pltpu
from jax.experimental.pallas import tpu_sc as plsc

N = 10000
D = 128
E = 320000
G = 64
NCORES = 2
NSUB = 16
NW = NCORES * NSUB
K = 128
GS = 20
NG = 4
NCH = NG * GS
NBUF = 2
NPAD = 10112
RPT = NPAD // NSUB
BN_EPS_K = 1e-5


def _sc_agg_body(h_hbm, col_hbm, row_hbm, zeros_hbm, out_hbm,
                 idxc_v, idxr_v, rows_v, gsem0, gsem1, isemc, isemr, accum_sh):
    c = lax.axis_index("c")
    s = lax.axis_index("s")
    wid = c * NSUB + s
    pltpu.sync_copy(zeros_hbm, accum_sh.at[pl.ds(s * RPT, RPT)])
    pltpu.sync_copy(col_hbm.at[wid, 0], idxc_v.at[0])
    pltpu.sync_copy(row_hbm.at[wid, 0], idxr_v.at[0])
    plsc.subcore_barrier()
    gsems = [gsem0, gsem1]
    ipend = [None, None]
    if NG > 1:
        ipend[1] = (
            pltpu.async_copy(col_hbm.at[wid, 1], idxc_v.at[1], isemc),
            pltpu.async_copy(row_hbm.at[wid, 1], idxr_v.at[1], isemr),
        )
    dpend = [None] * NCH

    def issue(j):
        g, b = divmod(j, GS)
        dpend[j] = pltpu.async_copy(h_hbm.at[idxc_v.at[g % 2, b]],
                                    rows_v.at[j % NBUF], gsems[j % NBUF])

    for j in range(NBUF):
        issue(j)
    for j in range(NCH):
        g, b = divmod(j, GS)
        dpend[j].wait()
        pltpu.sync_copy(rows_v.at[j % NBUF], accum_sh.at[idxr_v.at[g % 2, b]],
                        add=True)
        if (j + 1) % GS == 0 and (j + 1) // GS + 1 < NG:
            gg = (j + 1) // GS + 1
            ipend[gg % 2] = (
                pltpu.async_copy(col_hbm.at[wid, gg], idxc_v.at[gg % 2], isemc),
                pltpu.async_copy(row_hbm.at[wid, gg], idxr_v.at[gg % 2], isemr),
            )
        nxt = j + NBUF
        if nxt < NCH:
            ng = nxt // GS
            if ng > 0 and nxt % GS < NBUF and ipend[ng % 2] is not None:
                for d in ipend[ng % 2]:
                    d.wait()
                ipend[ng % 2] = None
            issue(nxt)
    plsc.subcore_barrier()
    pltpu.sync_copy(accum_sh.at[pl.ds(s * RPT, RPT)],
                    out_hbm.at[c].at[pl.ds(s * RPT, RPT)])


_sc_agg = pl.kernel(
    _sc_agg_body,
    out_type=jax.ShapeDtypeStruct((NCORES, NPAD, D), jnp.float32),
    mesh=plsc.VectorSubcoreMesh(core_axis_name="c", subcore_axis_name="s"),
    scratch_types=[
        pltpu.MemorySpace.VMEM((2, GS, K), jnp.int32),
        pltpu.MemorySpace.VMEM((2, GS, K), jnp.int32),
        pltpu.MemorySpace.VMEM((NBUF, K, D), jnp.float32),
        pltpu.SemaphoreType.DMA,
        pltpu.SemaphoreType.DMA,
        pltpu.SemaphoreType.DMA,
        pltpu.SemaphoreType.DMA,
        pltpu.MemorySpace.VMEM_SHARED((NPAD, D), jnp.float32),
    ],
)


def _bn_relu(z, gamma, beta):
    mean = jnp.mean(z, axis=0, keepdims=True)
    var = jnp.mean((z - mean) * (z - mean), axis=0, keepdims=True)
    zn = gamma * (z - mean) * lax.rsqrt(var + BN_EPS_K) + beta
    return jnp.maximum(zn, 0.0)


def _tc_layer_body(eps_ref, parts_ref, h_ref,
                   w0_ref, b0_ref, g0_ref, be0_ref,
                   w1_ref, b1_ref, gl_ref, bel_ref, out_ref):
    eps = eps_ref[0]
    pooled = (parts_ref[0, 0:N, :] + parts_ref[1, 0:N, :]
              + (1.0 + eps) * h_ref[...])
    z = lax.dot_general(pooled, w0_ref[...], (((1,), (1,)), ((), ())),
                        preferred_element_type=jnp.float32) + b0_ref[...]
    z = _bn_relu(z, g0_ref[...], be0_ref[...])
    z = lax.dot_general(z, w1_ref[...], (((1,), (1,)), ((), ())),
                        preferred_element_type=jnp.float32) + b1_ref[...]
    out_ref[...] = _bn_relu(z, gl_ref[...], bel_ref[...])


_tc_layer = pl.pallas_call(
    _tc_layer_body,
    out_shape=jax.ShapeDtypeStruct((N, D), jnp.float32),
    in_specs=[
        pl.BlockSpec(memory_space=pltpu.MemorySpace.SMEM),
    ] + [pl.BlockSpec(memory_space=pltpu.MemorySpace.VMEM)] * 10,
    out_specs=pl.BlockSpec(memory_space=pltpu.MemorySpace.VMEM),
)


def _tc_pool_body(batch_ref, h0_ref, h1_ref, h2_ref, h3_ref, h4_ref,
                  w_ref, b_ref, out_ref):
    b = jnp.broadcast_to(batch_ref[...], (G, N))
    gi = lax.broadcasted_iota(jnp.int32, (G, N), 0)
    p = (b == gi).astype(jnp.float32)
    counts = jnp.sum(p, axis=1, keepdims=True)
    inv = 1.0 / jnp.maximum(counts, 1.0)
    acc = jnp.zeros((G, D), jnp.float32)
    for l in range(5):
        h_ref = (h0_ref, h1_ref, h2_ref, h3_ref, h4_ref)[l]
        pooled = lax.dot_general(p, h_ref[...], (((1,), (0,)), ((), ())),
                                 preferred_element_type=jnp.float32) * inv
        acc = acc + lax.dot_general(pooled, w_ref[l],
                                    (((1,), (1,)), ((), ())),
                                    preferred_element_type=jnp.float32)
        acc = acc + b_ref[l]
    out_ref[...] = acc


_tc_pool = pl.pallas_call(
    _tc_pool_body,
    out_shape=jax.ShapeDtypeStruct((G, D), jnp.float32),
)


def kernel(x, params, edge_index, batch):
    row = edge_index[0]
    col = edge_index[1]
    pad = NW * NG * GS * K - E
    colp = jnp.concatenate([col, jnp.zeros((pad,), jnp.int32)]).reshape(NW, NG, GS, K)
    rowp = jnp.concatenate([row, jnp.full((pad,), N, jnp.int32)]).reshape(NW, NG, GS, K)
    colp = jnp.concatenate([colp, jnp.zeros((NW, 1, GS, K), jnp.int32)], axis=1)
    rowp = jnp.concatenate([rowp, jnp.full((NW, 1, GS, K), N, jnp.int32)], axis=1)
    zeros_init = jnp.zeros((RPT, D), jnp.float32)

    hs = [x]
    h = x
    for l in range(4):
        parts = _sc_agg(h, colp, rowp, zeros_init)
        mlp = params["mlp%d" % l]
        h = _tc_layer(
            params["eps"][l].reshape(1),
            parts, h,
            mlp["W0"], mlp["b0"].reshape(1, D),
            mlp["bn_g0"].reshape(1, D), mlp["bn_b0"].reshape(1, D),
            mlp["W1"], mlp["b1"].reshape(1, D),
            params["bn_g%d" % l].reshape(1, D),
            params["bn_b%d" % l].reshape(1, D),
        )
        hs.append(h)

    wstack = jnp.stack([params["pred%d_W" % l] for l in range(5)])
    bstack = jnp.stack([params["pred%d_b" % l] for l in range(5)]).reshape(5, 1, D)
    score = _tc_pool(batch.reshape(1, N), *hs, wstack, bstack)
    return score


# speedup vs baseline: 5.6978x; 3.4082x over previous
import functools

import jax
import jax.numpy as jnp
from jax import lax
from jax.experimental import pallas as pl
from jax.experimental.pallas import tpu as pltpu
from jax.experimental.pallas import tpu_sc as plsc

N = 10000
D = 128
E = 320000
G = 64
NCORES = 2
NSUB = 16
NW = NCORES * NSUB
K = 128
GS = 20
NG = 4
NCH = NG * GS
NBUF = 2
NPAD = 10112
RPT = NPAD // NSUB
BN_EPS_K = 1e-5


def _sc_agg_body(h_hbm, col_hbm, row_hbm, zeros_hbm, out_hbm,
                 idxc_v, idxr_v, rows_v, gsem0, gsem1, isemc, isemr, accum_sh):
    c = lax.axis_index("c")
    s = lax.axis_index("s")
    wid = c * NSUB + s
    pltpu.sync_copy(zeros_hbm, accum_sh.at[pl.ds(s * RPT, RPT)])
    pltpu.sync_copy(col_hbm.at[wid, 0], idxc_v.at[0])
    pltpu.sync_copy(row_hbm.at[wid, 0], idxr_v.at[0])
    plsc.subcore_barrier()
    gsems = [gsem0, gsem1]
    ipend = [None, None]
    if NG > 1:
        ipend[1] = (
            pltpu.async_copy(col_hbm.at[wid, 1], idxc_v.at[1], isemc),
            pltpu.async_copy(row_hbm.at[wid, 1], idxr_v.at[1], isemr),
        )
    dpend = [None] * NCH

    def issue(j):
        g, b = divmod(j, GS)
        dpend[j] = pltpu.async_copy(h_hbm.at[idxc_v.at[g % 2, b]],
                                    rows_v.at[j % NBUF], gsems[j % NBUF])

    for j in range(NBUF):
        issue(j)
    for j in range(NCH):
        g, b = divmod(j, GS)
        dpend[j].wait()
        pltpu.sync_copy(rows_v.at[j % NBUF], accum_sh.at[idxr_v.at[g % 2, b]],
                        add=True)
        if (j + 1) % GS == 0 and (j + 1) // GS + 1 < NG:
            gg = (j + 1) // GS + 1
            ipend[gg % 2] = (
                pltpu.async_copy(col_hbm.at[wid, gg], idxc_v.at[gg % 2], isemc),
                pltpu.async_copy(row_hbm.at[wid, gg], idxr_v.at[gg % 2], isemr),
            )
        nxt = j + NBUF
        if nxt < NCH:
            ng = nxt // GS
            if ng > 0 and nxt % GS < NBUF and ipend[ng % 2] is not None:
                for d in ipend[ng % 2]:
                    d.wait()
                ipend[ng % 2] = None
            issue(nxt)
    plsc.subcore_barrier()
    pltpu.sync_copy(accum_sh.at[pl.ds(s * RPT, RPT)],
                    out_hbm.at[c].at[pl.ds(s * RPT, RPT)])


_sc_agg = pl.kernel(
    _sc_agg_body,
    out_type=jax.ShapeDtypeStruct((NCORES, NPAD, D), jnp.float32),
    mesh=plsc.VectorSubcoreMesh(core_axis_name="c", subcore_axis_name="s"),
    scratch_types=[
        pltpu.MemorySpace.VMEM((2, GS, K), jnp.int32),
        pltpu.MemorySpace.VMEM((2, GS, K), jnp.int32),
        pltpu.MemorySpace.VMEM((NBUF, K, D), jnp.float32),
        pltpu.SemaphoreType.DMA,
        pltpu.SemaphoreType.DMA,
        pltpu.SemaphoreType.DMA,
        pltpu.SemaphoreType.DMA,
        pltpu.MemorySpace.VMEM_SHARED((NPAD, D), jnp.float32),
    ],
)


def _bn_relu(z, gamma, beta):
    mean = jnp.mean(z, axis=0, keepdims=True)
    var = jnp.mean((z - mean) * (z - mean), axis=0, keepdims=True)
    zn = gamma * (z - mean) * lax.rsqrt(var + BN_EPS_K) + beta
    return jnp.maximum(zn, 0.0)


def _tc_layer_body(eps_ref, parts_ref, h_ref,
                   w0_ref, b0_ref, g0_ref, be0_ref,
                   w1_ref, b1_ref, gl_ref, bel_ref, out_ref):
    eps = eps_ref[0]
    pooled = (parts_ref[0, 0:N, :] + parts_ref[1, 0:N, :]
              + (1.0 + eps) * h_ref[...])
    z = lax.dot_general(pooled, w0_ref[...], (((1,), (1,)), ((), ())),
                        preferred_element_type=jnp.float32) + b0_ref[...]
    z = _bn_relu(z, g0_ref[...], be0_ref[...])
    z = lax.dot_general(z, w1_ref[...], (((1,), (1,)), ((), ())),
                        preferred_element_type=jnp.float32) + b1_ref[...]
    out_ref[...] = _bn_relu(z, gl_ref[...], bel_ref[...])


_tc_layer = pl.pallas_call(
    _tc_layer_body,
    out_shape=jax.ShapeDtypeStruct((N, D), jnp.float32),
    in_specs=[
        pl.BlockSpec(memory_space=pltpu.MemorySpace.SMEM),
    ] + [pl.BlockSpec(memory_space=pltpu.MemorySpace.VMEM)] * 10,
    out_specs=pl.BlockSpec(memory_space=pltpu.MemorySpace.VMEM),
)


def _tc_pool_body(batch_ref, h0_ref, h1_ref, h2_ref, h3_ref, h4_ref,
                  w_ref, b_ref, out_ref):
    b = jnp.broadcast_to(batch_ref[...], (G, N))
    gi = lax.broadcasted_iota(jnp.int32, (G, N), 0)
    p = (b == gi).astype(jnp.float32)
    counts = jnp.sum(p, axis=1, keepdims=True)
    inv = 1.0 / jnp.maximum(counts, 1.0)
    acc = jnp.zeros((G, D), jnp.float32)
    for l in range(5):
        h_ref = (h0_ref, h1_ref, h2_ref, h3_ref, h4_ref)[l]
        pooled = lax.dot_general(p, h_ref[...], (((1,), (0,)), ((), ())),
                                 preferred_element_type=jnp.float32) * inv
        acc = acc + lax.dot_general(pooled, w_ref[l],
                                    (((1,), (1,)), ((), ())),
                                    preferred_element_type=jnp.float32)
        acc = acc + b_ref[l]
    out_ref[...] = acc


_tc_pool = pl.pallas_call(
    _tc_pool_body,
    out_shape=jax.ShapeDtypeStruct((G, D), jnp.float32),
)


def kernel(x, params, edge_index, batch):
    row = edge_index[0]
    col = edge_index[1]
    pad = NW * NG * GS * K - E
    pr = jnp.arange(pad, dtype=jnp.int32)
    colp = jnp.concatenate([col, pr % N]).reshape(NW, NG, GS, K)
    rowp = jnp.concatenate([row, N + pr % (NPAD - N)]).reshape(NW, NG, GS, K)
    zeros_init = jnp.zeros((RPT, D), jnp.float32)

    hs = [x]
    h = x
    for l in range(4):
        parts = _sc_agg(h, colp, rowp, zeros_init)
        mlp = params["mlp%d" % l]
        h = _tc_layer(
            params["eps"][l].reshape(1),
            parts, h,
            mlp["W0"], mlp["b0"].reshape(1, D),
            mlp["bn_g0"].reshape(1, D), mlp["bn_b0"].reshape(1, D),
            mlp["W1"], mlp["b1"].reshape(1, D),
            params["bn_g%d" % l].reshape(1, D),
            params["bn_b%d" % l].reshape(1, D),
        )
        hs.append(h)

    wstack = jnp.stack([params["pred%d_W" % l] for l in range(5)])
    bstack = jnp.stack([params["pred%d_b" % l] for l in range(5)]).reshape(5, 1, D)
    score = _tc_pool(batch.reshape(1, N), *hs, wstack, bstack)
    return score


# speedup vs baseline: 5.7837x; 1.0151x over previous
import functools

import jax
import jax.numpy as jnp
from jax import lax
from jax.experimental import pallas as pl
from jax.experimental.pallas import tpu as pltpu
from jax.experimental.pallas import tpu_sc as plsc

N = 10000
D = 128
E = 320000
G = 64
NCORES = 2
NSUB = 16
NW = NCORES * NSUB
K = 128
GS = 20
NG = 4
NCH = NG * GS
NBUF = 2
NPAD = 10112
RPT = NPAD // NSUB
BN_EPS_K = 1e-5


def _sc_agg_body(h_hbm, col_hbm, row_hbm, zeros_hbm, out_hbm,
                 idxc_v, idxr_v, rows_v, gsem0, gsem1, isemc, isemr, zsem,
                 accum_sh):
    c = lax.axis_index("c")
    s = lax.axis_index("s")
    wid = c * NSUB + s
    zinit = pltpu.async_copy(zeros_hbm, accum_sh.at[pl.ds(s * RPT, RPT)],
                             zsem)
    pltpu.sync_copy(col_hbm.at[wid, 0], idxc_v.at[0])
    pltpu.sync_copy(row_hbm.at[wid, 0], idxr_v.at[0])
    gsems = [gsem0, gsem1]
    ipend = [None, None]
    if NG > 1:
        ipend[1] = (
            pltpu.async_copy(col_hbm.at[wid, 1], idxc_v.at[1], isemc),
            pltpu.async_copy(row_hbm.at[wid, 1], idxr_v.at[1], isemr),
        )
    dpend = [None] * NCH

    def issue(j):
        g, b = divmod(j, GS)
        dpend[j] = pltpu.async_copy(h_hbm.at[idxc_v.at[g % 2, b]],
                                    rows_v.at[j % NBUF], gsems[j % NBUF])

    for j in range(NBUF):
        issue(j)
    zinit.wait()
    plsc.subcore_barrier()
    for j in range(NCH):
        g, b = divmod(j, GS)
        dpend[j].wait()
        pltpu.sync_copy(rows_v.at[j % NBUF], accum_sh.at[idxr_v.at[g % 2, b]],
                        add=True)
        if (j + 1) % GS == 0 and (j + 1) // GS + 1 < NG:
            gg = (j + 1) // GS + 1
            ipend[gg % 2] = (
                pltpu.async_copy(col_hbm.at[wid, gg], idxc_v.at[gg % 2], isemc),
                pltpu.async_copy(row_hbm.at[wid, gg], idxr_v.at[gg % 2], isemr),
            )
        nxt = j + NBUF
        if nxt < NCH:
            ng = nxt // GS
            if ng > 0 and nxt % GS < NBUF and ipend[ng % 2] is not None:
                for d in ipend[ng % 2]:
                    d.wait()
                ipend[ng % 2] = None
            issue(nxt)
    plsc.subcore_barrier()
    pltpu.sync_copy(accum_sh.at[pl.ds(s * RPT, RPT)],
                    out_hbm.at[c].at[pl.ds(s * RPT, RPT)])


_sc_agg = pl.kernel(
    _sc_agg_body,
    out_type=jax.ShapeDtypeStruct((NCORES, NPAD, D), jnp.float32),
    mesh=plsc.VectorSubcoreMesh(core_axis_name="c", subcore_axis_name="s"),
    scratch_types=[
        pltpu.MemorySpace.VMEM((2, GS, K), jnp.int32),
        pltpu.MemorySpace.VMEM((2, GS, K), jnp.int32),
        pltpu.MemorySpace.VMEM((NBUF, K, D), jnp.float32),
        pltpu.SemaphoreType.DMA,
        pltpu.SemaphoreType.DMA,
        pltpu.SemaphoreType.DMA,
        pltpu.SemaphoreType.DMA,
        pltpu.SemaphoreType.DMA,
        pltpu.MemorySpace.VMEM_SHARED((NPAD, D), jnp.float32),
    ],
)


def _bn_relu(z, gamma, beta):
    mean = jnp.mean(z, axis=0, keepdims=True)
    var = jnp.mean((z - mean) * (z - mean), axis=0, keepdims=True)
    zn = gamma * (z - mean) * lax.rsqrt(var + BN_EPS_K) + beta
    return jnp.maximum(zn, 0.0)


def _tc_layer_body(eps_ref, parts_ref, h_ref,
                   w0_ref, b0_ref, g0_ref, be0_ref,
                   w1_ref, b1_ref, gl_ref, bel_ref,
                   pa_ref, wp_ref, bp_ref, out_ref, contrib_ref):
    eps = eps_ref[0]
    pooled_h = lax.dot_general(pa_ref[...], h_ref[...],
                               (((1,), (0,)), ((), ())),
                               preferred_element_type=jnp.float32)
    contrib_ref[...] = lax.dot_general(pooled_h, wp_ref[...],
                                       (((1,), (1,)), ((), ())),
                                       preferred_element_type=jnp.float32
                                       ) + bp_ref[...]
    pooled = (parts_ref[0, 0:N, :] + parts_ref[1, 0:N, :]
              + (1.0 + eps) * h_ref[...])
    z = lax.dot_general(pooled, w0_ref[...], (((1,), (1,)), ((), ())),
                        preferred_element_type=jnp.float32) + b0_ref[...]
    z = _bn_relu(z, g0_ref[...], be0_ref[...])
    z = lax.dot_general(z, w1_ref[...], (((1,), (1,)), ((), ())),
                        preferred_element_type=jnp.float32) + b1_ref[...]
    out_ref[...] = _bn_relu(z, gl_ref[...], bel_ref[...])


_tc_layer = pl.pallas_call(
    _tc_layer_body,
    out_shape=(jax.ShapeDtypeStruct((N, D), jnp.float32),
               jax.ShapeDtypeStruct((G, D), jnp.float32)),
    in_specs=[
        pl.BlockSpec(memory_space=pltpu.MemorySpace.SMEM),
    ] + [pl.BlockSpec(memory_space=pltpu.MemorySpace.VMEM)] * 13,
    out_specs=(pl.BlockSpec(memory_space=pltpu.MemorySpace.VMEM),
               pl.BlockSpec(memory_space=pltpu.MemorySpace.VMEM)),
)


def _tc_poolmat_body(batch_ref, pa_ref):
    b = jnp.broadcast_to(batch_ref[...], (G, N))
    gi = lax.broadcasted_iota(jnp.int32, (G, N), 0)
    p = (b == gi).astype(jnp.float32)
    counts = jnp.sum(p, axis=1, keepdims=True)
    pa_ref[...] = p * (1.0 / jnp.maximum(counts, 1.0))


_tc_poolmat = pl.pallas_call(
    _tc_poolmat_body,
    out_shape=jax.ShapeDtypeStruct((G, N), jnp.float32),
)


def _tc_pool_body(pa_ref, h_ref, w_ref, b_ref,
                  c0_ref, c1_ref, c2_ref, c3_ref, out_ref):
    pooled = lax.dot_general(pa_ref[...], h_ref[...], (((1,), (0,)), ((), ())),
                             preferred_element_type=jnp.float32)
    acc = lax.dot_general(pooled, w_ref[...], (((1,), (1,)), ((), ())),
                          preferred_element_type=jnp.float32) + b_ref[...]
    out_ref[...] = (acc + c0_ref[...] + c1_ref[...]
                    + c2_ref[...] + c3_ref[...])


_tc_pool = pl.pallas_call(
    _tc_pool_body,
    out_shape=jax.ShapeDtypeStruct((G, D), jnp.float32),
)


def kernel(x, params, edge_index, batch):
    row = edge_index[0]
    col = edge_index[1]
    pad = NW * NG * GS * K - E
    pr = jnp.arange(pad, dtype=jnp.int32)
    colp = jnp.concatenate([col, pr % N]).reshape(NW, NG, GS, K)
    rowp = jnp.concatenate([row, N + pr % (NPAD - N)]).reshape(NW, NG, GS, K)
    zeros_init = jnp.zeros((RPT, D), jnp.float32)

    pa = _tc_poolmat(batch.reshape(1, N))
    contribs = []
    h = x
    for l in range(4):
        parts = _sc_agg(h, colp, rowp, zeros_init)
        mlp = params["mlp%d" % l]
        h, contrib = _tc_layer(
            params["eps"][l].reshape(1),
            parts, h,
            mlp["W0"], mlp["b0"].reshape(1, D),
            mlp["bn_g0"].reshape(1, D), mlp["bn_b0"].reshape(1, D),
            mlp["W1"], mlp["b1"].reshape(1, D),
            params["bn_g%d" % l].reshape(1, D),
            params["bn_b%d" % l].reshape(1, D),
            pa, params["pred%d_W" % l], params["pred%d_b" % l].reshape(1, D),
        )
        contribs.append(contrib)

    score = _tc_pool(pa, h, params["pred4_W"], params["pred4_b"].reshape(1, D),
                     *contribs)
    return score


# speedup vs baseline: 5.8771x; 1.0162x over previous
import functools

import jax
import jax.numpy as jnp
from jax import lax
from jax.experimental import pallas as pl
from jax.experimental.pallas import tpu as pltpu
from jax.experimental.pallas import tpu_sc as plsc

N = 10000
D = 128
E = 320000
G = 64
NCORES = 2
NSUB = 16
NW = NCORES * NSUB
K = 128
GS = 20
NG = 4
NCH = NG * GS
NBUF = 2
NPAD = 10112
RPT = NPAD // NSUB
BN_EPS_K = 1e-5


def _sc_agg_body(h_hbm, col_hbm, row_hbm, zeros_hbm, out_hbm,
                 idxc_v, idxr_v, rows_v, gsem0, gsem1, isemc, isemr, zsem,
                 accum_sh):
    c = lax.axis_index("c")
    s = lax.axis_index("s")
    wid = c * NSUB + s
    zinit = pltpu.async_copy(zeros_hbm, accum_sh.at[pl.ds(s * RPT, RPT)],
                             zsem)
    pltpu.sync_copy(col_hbm.at[wid, 0], idxc_v.at[0])
    pltpu.sync_copy(row_hbm.at[wid, 0], idxr_v.at[0])
    gsems = [gsem0, gsem1]
    ipend = [None, None]
    if NG > 1:
        ipend[1] = (
            pltpu.async_copy(col_hbm.at[wid, 1], idxc_v.at[1], isemc),
            pltpu.async_copy(row_hbm.at[wid, 1], idxr_v.at[1], isemr),
        )
    dpend = [None] * NCH

    def issue(j):
        g, b = divmod(j, GS)
        dpend[j] = pltpu.async_copy(h_hbm.at[idxc_v.at[g % 2, b]],
                                    rows_v.at[j % NBUF], gsems[j % NBUF])

    for j in range(NBUF):
        issue(j)
    zinit.wait()
    plsc.subcore_barrier()
    for j in range(NCH):
        g, b = divmod(j, GS)
        dpend[j].wait()
        pltpu.sync_copy(rows_v.at[j % NBUF], accum_sh.at[idxr_v.at[g % 2, b]],
                        add=True)
        if (j + 1) % GS == 0 and (j + 1) // GS + 1 < NG:
            gg = (j + 1) // GS + 1
            ipend[gg % 2] = (
                pltpu.async_copy(col_hbm.at[wid, gg], idxc_v.at[gg % 2], isemc),
                pltpu.async_copy(row_hbm.at[wid, gg], idxr_v.at[gg % 2], isemr),
            )
        nxt = j + NBUF
        if nxt < NCH:
            ng = nxt // GS
            if ng > 0 and nxt % GS < NBUF and ipend[ng % 2] is not None:
                for d in ipend[ng % 2]:
                    d.wait()
                ipend[ng % 2] = None
            issue(nxt)
    plsc.subcore_barrier()
    pltpu.sync_copy(accum_sh.at[pl.ds(s * RPT, RPT)],
                    out_hbm.at[c].at[pl.ds(s * RPT, RPT)])


_sc_agg = pl.kernel(
    _sc_agg_body,
    out_type=jax.ShapeDtypeStruct((NCORES, NPAD, D), jnp.float32),
    mesh=plsc.VectorSubcoreMesh(core_axis_name="c", subcore_axis_name="s"),
    scratch_types=[
        pltpu.MemorySpace.VMEM((2, GS, K), jnp.int32),
        pltpu.MemorySpace.VMEM((2, GS, K), jnp.int32),
        pltpu.MemorySpace.VMEM((NBUF, K, D), jnp.float32),
        pltpu.SemaphoreType.DMA,
        pltpu.SemaphoreType.DMA,
        pltpu.SemaphoreType.DMA,
        pltpu.SemaphoreType.DMA,
        pltpu.SemaphoreType.DMA,
        pltpu.MemorySpace.VMEM_SHARED((NPAD, D), jnp.float32),
    ],
)


def _bn_relu(z, gamma, beta):
    mean = jnp.mean(z, axis=0, keepdims=True)
    msq = jnp.mean(z * z, axis=0, keepdims=True)
    var = msq - mean * mean
    zn = gamma * (z - mean) * lax.rsqrt(var + BN_EPS_K) + beta
    return jnp.maximum(zn, 0.0)


def _tc_layer_body(eps_ref, parts_ref, h_ref,
                   w0_ref, b0_ref, g0_ref, be0_ref,
                   w1_ref, b1_ref, gl_ref, bel_ref,
                   pa_ref, wp_ref, bp_ref, out_ref, contrib_ref):
    eps = eps_ref[0]
    pooled_h = lax.dot_general(pa_ref[...], h_ref[...],
                               (((1,), (0,)), ((), ())),
                               preferred_element_type=jnp.float32)
    contrib_ref[...] = lax.dot_general(pooled_h, wp_ref[...],
                                       (((1,), (1,)), ((), ())),
                                       preferred_element_type=jnp.float32
                                       ) + bp_ref[...]
    pooled = (parts_ref[0, 0:N, :] + parts_ref[1, 0:N, :]
              + (1.0 + eps) * h_ref[...])
    z = lax.dot_general(pooled, w0_ref[...], (((1,), (1,)), ((), ())),
                        preferred_element_type=jnp.float32) + b0_ref[...]
    z = _bn_relu(z, g0_ref[...], be0_ref[...])
    z = lax.dot_general(z, w1_ref[...], (((1,), (1,)), ((), ())),
                        preferred_element_type=jnp.float32) + b1_ref[...]
    out_ref[...] = _bn_relu(z, gl_ref[...], bel_ref[...])


_tc_layer = pl.pallas_call(
    _tc_layer_body,
    out_shape=(jax.ShapeDtypeStruct((N, D), jnp.float32),
               jax.ShapeDtypeStruct((G, D), jnp.float32)),
    in_specs=[
        pl.BlockSpec(memory_space=pltpu.MemorySpace.SMEM),
    ] + [pl.BlockSpec(memory_space=pltpu.MemorySpace.VMEM)] * 13,
    out_specs=(pl.BlockSpec(memory_space=pltpu.MemorySpace.VMEM),
               pl.BlockSpec(memory_space=pltpu.MemorySpace.VMEM)),
)


def _tc_poolmat_body(batch_ref, pa_ref):
    b = jnp.broadcast_to(batch_ref[...], (G, N))
    gi = lax.broadcasted_iota(jnp.int32, (G, N), 0)
    p = (b == gi).astype(jnp.float32)
    counts = jnp.sum(p, axis=1, keepdims=True)
    pa_ref[...] = p * (1.0 / jnp.maximum(counts, 1.0))


_tc_poolmat = pl.pallas_call(
    _tc_poolmat_body,
    out_shape=jax.ShapeDtypeStruct((G, N), jnp.float32),
)


def _tc_pool_body(pa_ref, h_ref, w_ref, b_ref,
                  c0_ref, c1_ref, c2_ref, c3_ref, out_ref):
    pooled = lax.dot_general(pa_ref[...], h_ref[...], (((1,), (0,)), ((), ())),
                             preferred_element_type=jnp.float32)
    acc = lax.dot_general(pooled, w_ref[...], (((1,), (1,)), ((), ())),
                          preferred_element_type=jnp.float32) + b_ref[...]
    out_ref[...] = (acc + c0_ref[...] + c1_ref[...]
                    + c2_ref[...] + c3_ref[...])


_tc_pool = pl.pallas_call(
    _tc_pool_body,
    out_shape=jax.ShapeDtypeStruct((G, D), jnp.float32),
)


def kernel(x, params, edge_index, batch):
    row = edge_index[0]
    col = edge_index[1]
    pad = NW * NG * GS * K - E
    pr = jnp.arange(pad, dtype=jnp.int32)
    colp = jnp.concatenate([col, pr % N]).reshape(NW, NG, GS, K)
    rowp = jnp.concatenate([row, N + pr % (NPAD - N)]).reshape(NW, NG, GS, K)
    zeros_init = jnp.zeros((RPT, D), jnp.float32)

    pa = _tc_poolmat(batch.reshape(1, N))
    contribs = []
    h = x
    for l in range(4):
        parts = _sc_agg(h, colp, rowp, zeros_init)
        mlp = params["mlp%d" % l]
        h, contrib = _tc_layer(
            params["eps"][l].reshape(1),
            parts, h,
            mlp["W0"], mlp["b0"].reshape(1, D),
            mlp["bn_g0"].reshape(1, D), mlp["bn_b0"].reshape(1, D),
            mlp["W1"], mlp["b1"].reshape(1, D),
            params["bn_g%d" % l].reshape(1, D),
            params["bn_b%d" % l].reshape(1, D),
            pa, params["pred%d_W" % l], params["pred%d_b" % l].reshape(1, D),
        )
        contribs.append(contrib)

    score = _tc_pool(pa, h, params["pred4_W"], params["pred4_b"].reshape(1, D),
                     *contribs)
    return score
